# baseline (speedup 1.0000x reference)
import functools

import jax
import jax.numpy as jnp
from jax.experimental import pallas as pl
from jax.experimental.pallas import tpu as pltpu

B = 4
C_IN = 256
C_SH = 64
N = 5000
HEAD_CLS = (2, 1, 3, 2, 1, 3)
C_OUT = sum(HEAD_CLS)
C_MID = C_SH * len(HEAD_CLS)
EPS = 1e-5


def _head_body(x_ref, a_ref, ba_ref, w_ref, bw_ref, c_ref, bc_ref, o_ref):
    x = x_ref[0].astype(jnp.bfloat16)
    y = jnp.dot(x, a_ref[...], preferred_element_type=jnp.float32)
    y = jnp.maximum(y + ba_ref[...], 0.0).astype(jnp.bfloat16)
    h = jnp.dot(y, w_ref[...], preferred_element_type=jnp.float32)
    h = jnp.maximum(h + bw_ref[...], 0.0).astype(jnp.bfloat16)
    o = jnp.dot(h, c_ref[...], preferred_element_type=jnp.float32)
    o_ref[0] = o + bc_ref[...]


@functools.partial(jax.jit, static_argnames=())
def kernel(ct_feat, sh_W, sh_b, sh_g, sh_be,
           reg_W1, reg_b1, reg_g1, reg_be1, reg_W2, reg_b2,
           height_W1, height_b1, height_g1, height_be1, height_W2, height_b2,
           dim_W1, dim_b1, dim_g1, dim_be1, dim_W2, dim_b2,
           rot_W1, rot_b1, rot_g1, rot_be1, rot_W2, rot_b2,
           iou_W1, iou_b1, iou_g1, iou_be1, iou_W2, iou_b2,
           hm_W1, hm_b1, hm_g1, hm_be1, hm_W2, hm_b2):
    inv_s = 1.0 / jnp.sqrt(1.0 + EPS)

    at = (sh_W * (sh_g * inv_s)[:, None]).T
    ba = sh_b * sh_g * inv_s + sh_be

    heads = (
        (reg_W1, reg_b1, reg_g1, reg_be1, reg_W2, reg_b2),
        (height_W1, height_b1, height_g1, height_be1, height_W2, height_b2),
        (dim_W1, dim_b1, dim_g1, dim_be1, dim_W2, dim_b2),
        (rot_W1, rot_b1, rot_g1, rot_be1, rot_W2, rot_b2),
        (iou_W1, iou_b1, iou_g1, iou_be1, iou_W2, iou_b2),
        (hm_W1, hm_b1, hm_g1, hm_be1, hm_W2, hm_b2),
    )

    wt = jnp.concatenate(
        [(w1 * (g1 * inv_s)[:, None]).T for (w1, b1, g1, be1, _, _) in heads],
        axis=1)
    bw = jnp.concatenate(
        [b1 * g1 * inv_s + be1 for (_, b1, g1, be1, _, _) in heads], axis=0)

    c_blocks = []
    bc_rows = []
    for i, (_, _, _, _, w2, b2) in enumerate(heads):
        cls = w2.shape[0]
        off = sum(HEAD_CLS[:i])
        c_blocks.append(jnp.pad(w2.T, ((0, 0), (off, C_OUT - off - cls))))
        bc_rows.append(b2)
    ct = jnp.concatenate(c_blocks, axis=0)
    bc = jnp.concatenate(bc_rows, axis=0)

    xt = jnp.swapaxes(ct_feat, 1, 2)
    NR = 1000
    out = pl.pallas_call(
        _head_body,
        grid=(B, N // NR),
        in_specs=[
            pl.BlockSpec((1, NR, C_IN), lambda b, j: (b, j, 0)),
            pl.BlockSpec((C_IN, C_SH), lambda b, j: (0, 0)),
            pl.BlockSpec((1, C_SH), lambda b, j: (0, 0)),
            pl.BlockSpec((C_SH, C_MID), lambda b, j: (0, 0)),
            pl.BlockSpec((1, C_MID), lambda b, j: (0, 0)),
            pl.BlockSpec((C_MID, C_OUT), lambda b, j: (0, 0)),
            pl.BlockSpec((1, C_OUT), lambda b, j: (0, 0)),
        ],
        out_specs=pl.BlockSpec((1, NR, C_OUT), lambda b, j: (b, j, 0)),
        out_shape=jax.ShapeDtypeStruct((B, N, C_OUT), jnp.float32),
        compiler_params=pltpu.CompilerParams(
            dimension_semantics=("parallel", "parallel")),
    )(xt, at.astype(jnp.bfloat16), ba[None, :],
      wt.astype(jnp.bfloat16), bw[None, :],
      ct.astype(jnp.bfloat16), bc[None, :])
    return jnp.swapaxes(out, 1, 2)


# speedup vs baseline: 1.8248x; 1.8248x over previous
import functools

import jax
import jax.numpy as jnp
from jax.experimental import pallas as pl
from jax.experimental.pallas import tpu as pltpu

B = 4
C_IN = 256
C_SH = 64
N = 5000
HEAD_CLS = (2, 1, 3, 2, 1, 3)
C_OUT = sum(HEAD_CLS)
C_MID = C_SH * len(HEAD_CLS)
EPS = 1e-5
_DN = (((1,), (1,)), ((), ()))


def _head_body(x_ref, shW_ref, shb_ref, shg_ref, shbe_ref,
               w1_refs, b1_refs, g1_refs, be1_refs, w2_refs, b2_refs,
               o_ref):
    inv_s = 1.0 / jnp.sqrt(1.0 + EPS)
    x = x_ref[0].astype(jnp.bfloat16)

    y = jax.lax.dot_general(x, shW_ref[...].astype(jnp.bfloat16), _DN,
                            preferred_element_type=jnp.float32)
    sg = (shg_ref[...] * inv_s)[None, :]
    sb = (shb_ref[...] * shg_ref[...] * inv_s + shbe_ref[...])[None, :]
    y = jnp.maximum(y * sg + sb, 0.0).astype(jnp.bfloat16)

    w1 = jnp.concatenate([r[...] for r in w1_refs], axis=0).astype(jnp.bfloat16)
    g1 = jnp.concatenate([r[...] for r in g1_refs], axis=0)
    b1 = jnp.concatenate([r[...] for r in b1_refs], axis=0)
    be1 = jnp.concatenate([r[...] for r in be1_refs], axis=0)
    h = jax.lax.dot_general(y, w1, _DN, preferred_element_type=jnp.float32)
    h = jnp.maximum(h * (g1 * inv_s)[None, :]
                    + (b1 * g1 * inv_s + be1)[None, :], 0.0)
    h = h.astype(jnp.bfloat16)

    c_rows = []
    for i, r in enumerate(w2_refs):
        c_rows.append(jnp.pad(r[...], ((0, 0), (C_SH * i,
                                                C_MID - C_SH * (i + 1)))))
    c2 = jnp.concatenate(c_rows, axis=0).astype(jnp.bfloat16)
    b2 = jnp.concatenate([r[...] for r in b2_refs], axis=0)
    o = jax.lax.dot_general(h, c2, _DN, preferred_element_type=jnp.float32)
    o_ref[0] = o + b2[None, :]


@functools.partial(jax.jit, static_argnames=())
def kernel(ct_feat, sh_W, sh_b, sh_g, sh_be,
           reg_W1, reg_b1, reg_g1, reg_be1, reg_W2, reg_b2,
           height_W1, height_b1, height_g1, height_be1, height_W2, height_b2,
           dim_W1, dim_b1, dim_g1, dim_be1, dim_W2, dim_b2,
           rot_W1, rot_b1, rot_g1, rot_be1, rot_W2, rot_b2,
           iou_W1, iou_b1, iou_g1, iou_be1, iou_W2, iou_b2,
           hm_W1, hm_b1, hm_g1, hm_be1, hm_W2, hm_b2):
    w1s = (reg_W1, height_W1, dim_W1, rot_W1, iou_W1, hm_W1)
    b1s = (reg_b1, height_b1, dim_b1, rot_b1, iou_b1, hm_b1)
    g1s = (reg_g1, height_g1, dim_g1, rot_g1, iou_g1, hm_g1)
    be1s = (reg_be1, height_be1, dim_be1, rot_be1, iou_be1, hm_be1)
    w2s = (reg_W2, height_W2, dim_W2, rot_W2, iou_W2, hm_W2)
    b2s = (reg_b2, height_b2, dim_b2, rot_b2, iou_b2, hm_b2)

    xt = jnp.swapaxes(ct_feat, 1, 2)

    full = lambda s: pl.BlockSpec(s, lambda b: tuple(0 for _ in s))
    vec = lambda n: pl.BlockSpec((n,), lambda b: (0,))

    def body(x_ref, shW, shb, shg, shbe,
             w1a, w1b, w1c, w1d, w1e, w1f,
             b1a, b1b, b1c, b1d, b1e, b1f,
             g1a, g1b, g1c, g1d, g1e, g1f,
             be1a, be1b, be1c, be1d, be1e, be1f,
             w2a, w2b, w2c, w2d, w2e, w2f,
             b2a, b2b, b2c, b2d, b2e, b2f,
             o_ref):
        _head_body(x_ref, shW, shb, shg, shbe,
                   (w1a, w1b, w1c, w1d, w1e, w1f),
                   (b1a, b1b, b1c, b1d, b1e, b1f),
                   (g1a, g1b, g1c, g1d, g1e, g1f),
                   (be1a, be1b, be1c, be1d, be1e, be1f),
                   (w2a, w2b, w2c, w2d, w2e, w2f),
                   (b2a, b2b, b2c, b2d, b2e, b2f),
                   o_ref)

    in_specs = ([pl.BlockSpec((1, N, C_IN), lambda b: (b, 0, 0)),
                 full((C_SH, C_IN)), vec(C_SH), vec(C_SH), vec(C_SH)]
                + [full((C_SH, C_SH))] * 6
                + [vec(C_SH)] * 18
                + [full((cls, C_SH)) for cls in HEAD_CLS]
                + [vec(cls) for cls in HEAD_CLS])

    out = pl.pallas_call(
        body,
        grid=(B,),
        in_specs=in_specs,
        out_specs=pl.BlockSpec((1, N, C_OUT), lambda b: (b, 0, 0)),
        out_shape=jax.ShapeDtypeStruct((B, N, C_OUT), jnp.float32),
        compiler_params=pltpu.CompilerParams(
            dimension_semantics=("parallel",)),
    )(xt, sh_W, sh_b, sh_g, sh_be,
      *w1s, *b1s, *g1s, *be1s, *w2s, *b2s)
    return jnp.swapaxes(out, 1, 2)


# speedup vs baseline: 2.1871x; 1.1985x over previous
import functools

import jax
import jax.numpy as jnp
from jax.experimental import pallas as pl
from jax.experimental.pallas import tpu as pltpu

B = 4
C_IN = 256
C_SH = 64
N = 5000
HEAD_CLS = (2, 1, 3, 2, 1, 3)
C_OUT = sum(HEAD_CLS)
C_MID = C_SH * len(HEAD_CLS)
EPS = 1e-5
_DN = (((1,), (1,)), ((), ()))


def _head_body(x_ref, shW_ref, shb_ref, shg_ref, shbe_ref,
               w1_refs, b1_refs, g1_refs, be1_refs, w2_refs, b2_refs,
               o_ref):
    inv_s = 1.0 / jnp.sqrt(1.0 + EPS)
    x = x_ref[...].astype(jnp.bfloat16)

    y = jax.lax.dot_general(x, shW_ref[...].astype(jnp.bfloat16), _DN,
                            preferred_element_type=jnp.float32)
    sg = (shg_ref[...] * inv_s)[None, :]
    sb = (shb_ref[...] * shg_ref[...] * inv_s + shbe_ref[...])[None, :]
    y = jnp.maximum(y * sg + sb, 0.0).astype(jnp.bfloat16)

    w1 = jnp.concatenate([r[...] for r in w1_refs], axis=0).astype(jnp.bfloat16)
    g1 = jnp.concatenate([r[...] for r in g1_refs], axis=0)
    b1 = jnp.concatenate([r[...] for r in b1_refs], axis=0)
    be1 = jnp.concatenate([r[...] for r in be1_refs], axis=0)
    h = jax.lax.dot_general(y, w1, _DN, preferred_element_type=jnp.float32)
    h = jnp.maximum(h * (g1 * inv_s)[None, :]
                    + (b1 * g1 * inv_s + be1)[None, :], 0.0)
    h = h.astype(jnp.bfloat16)

    c_rows = []
    for i, r in enumerate(w2_refs):
        c_rows.append(jnp.pad(r[...], ((0, 0), (C_SH * i,
                                                C_MID - C_SH * (i + 1)))))
    c2 = jnp.concatenate(c_rows, axis=0).astype(jnp.bfloat16)
    b2 = jnp.concatenate([r[...] for r in b2_refs], axis=0)
    o = jax.lax.dot_general(h, c2, _DN, preferred_element_type=jnp.float32)
    o = o + b2[None, :]
    o_ref[:, 0, 0, :] = jnp.transpose(o, (1, 0))


@functools.partial(jax.jit, static_argnames=())
def kernel(ct_feat, sh_W, sh_b, sh_g, sh_be,
           reg_W1, reg_b1, reg_g1, reg_be1, reg_W2, reg_b2,
           height_W1, height_b1, height_g1, height_be1, height_W2, height_b2,
           dim_W1, dim_b1, dim_g1, dim_be1, dim_W2, dim_b2,
           rot_W1, rot_b1, rot_g1, rot_be1, rot_W2, rot_b2,
           iou_W1, iou_b1, iou_g1, iou_be1, iou_W2, iou_b2,
           hm_W1, hm_b1, hm_g1, hm_be1, hm_W2, hm_b2):
    w1s = (reg_W1, height_W1, dim_W1, rot_W1, iou_W1, hm_W1)
    b1s = (reg_b1, height_b1, dim_b1, rot_b1, iou_b1, hm_b1)
    g1s = (reg_g1, height_g1, dim_g1, rot_g1, iou_g1, hm_g1)
    be1s = (reg_be1, height_be1, dim_be1, rot_be1, iou_be1, hm_be1)
    w2s = (reg_W2, height_W2, dim_W2, rot_W2, iou_W2, hm_W2)
    b2s = (reg_b2, height_b2, dim_b2, rot_b2, iou_b2, hm_b2)

    xt = jnp.swapaxes(ct_feat, 1, 2).reshape(B * N, C_IN)

    full = lambda s: pl.BlockSpec(s, lambda b: tuple(0 for _ in s))
    vec = lambda n: pl.BlockSpec((n,), lambda b: (0,))

    def body(x_ref, shW, shb, shg, shbe,
             w1a, w1b, w1c, w1d, w1e, w1f,
             b1a, b1b, b1c, b1d, b1e, b1f,
             g1a, g1b, g1c, g1d, g1e, g1f,
             be1a, be1b, be1c, be1d, be1e, be1f,
             w2a, w2b, w2c, w2d, w2e, w2f,
             b2a, b2b, b2c, b2d, b2e, b2f,
             o_ref):
        _head_body(x_ref, shW, shb, shg, shbe,
                   (w1a, w1b, w1c, w1d, w1e, w1f),
                   (b1a, b1b, b1c, b1d, b1e, b1f),
                   (g1a, g1b, g1c, g1d, g1e, g1f),
                   (be1a, be1b, be1c, be1d, be1e, be1f),
                   (w2a, w2b, w2c, w2d, w2e, w2f),
                   (b2a, b2b, b2c, b2d, b2e, b2f),
                   o_ref)

    in_specs = ([pl.BlockSpec((N, C_IN), lambda b: (b, 0)),
                 full((C_SH, C_IN)), vec(C_SH), vec(C_SH), vec(C_SH)]
                + [full((C_SH, C_SH))] * 6
                + [vec(C_SH)] * 18
                + [full((cls, C_SH)) for cls in HEAD_CLS]
                + [vec(cls) for cls in HEAD_CLS])

    out = pl.pallas_call(
        body,
        grid=(B,),
        in_specs=in_specs,
        out_specs=pl.BlockSpec((C_OUT, 1, 1, N), lambda b: (0, b, 0, 0)),
        out_shape=jax.ShapeDtypeStruct((C_OUT, B, 1, N), jnp.float32),
        compiler_params=pltpu.CompilerParams(
            dimension_semantics=("parallel",)),
    )(xt, sh_W, sh_b, sh_g, sh_be,
      *w1s, *b1s, *g1s, *be1s, *w2s, *b2s)
    return jnp.swapaxes(out.reshape(C_OUT, B, N), 0, 1)


# speedup vs baseline: 2.3398x; 1.0698x over previous
import functools

import jax
import jax.numpy as jnp
from jax.experimental import pallas as pl
from jax.experimental.pallas import tpu as pltpu

B = 4
C_IN = 256
C_SH = 64
N = 5000
HEAD_CLS = (2, 1, 3, 2, 1, 3)
C_OUT = sum(HEAD_CLS)
C_MID = C_SH * len(HEAD_CLS)
EPS = 1e-5
_DN = (((1,), (1,)), ((), ()))
CHUNK = 1250


def _head_body(x_ref, shW_ref, shb_ref, shg_ref, shbe_ref,
               w1_refs, b1_refs, g1_refs, be1_refs, w2_refs, b2_refs,
               o_ref):
    inv_s = 1.0 / jnp.sqrt(1.0 + EPS)

    shW = shW_ref[...].astype(jnp.bfloat16)
    sg = (shg_ref[...] * inv_s)[None, :]
    sb = (shb_ref[...] * shg_ref[...] * inv_s + shbe_ref[...])[None, :]
    w1 = jnp.concatenate([r[...] for r in w1_refs], axis=0).astype(jnp.bfloat16)
    g1 = jnp.concatenate([r[...] for r in g1_refs], axis=0)
    b1 = jnp.concatenate([r[...] for r in b1_refs], axis=0)
    be1 = jnp.concatenate([r[...] for r in be1_refs], axis=0)
    sg1 = (g1 * inv_s)[None, :]
    sb1 = (b1 * g1 * inv_s + be1)[None, :]
    c_rows = []
    for i, r in enumerate(w2_refs):
        c_rows.append(jnp.pad(r[...], ((0, 0), (C_SH * i,
                                                C_MID - C_SH * (i + 1)))))
    c2 = jnp.concatenate(c_rows, axis=0).astype(jnp.bfloat16)
    b2 = jnp.concatenate([r[...] for r in b2_refs], axis=0)

    for c in range(N // CHUNK):
        x = x_ref[pl.ds(c * CHUNK, CHUNK), :].astype(jnp.bfloat16)
        y = jax.lax.dot_general(x, shW, _DN,
                                preferred_element_type=jnp.float32)
        y = jnp.maximum(y * sg + sb, 0.0).astype(jnp.bfloat16)
        h = jax.lax.dot_general(y, w1, _DN,
                                preferred_element_type=jnp.float32)
        h = jnp.maximum(h * sg1 + sb1, 0.0).astype(jnp.bfloat16)
        o = jax.lax.dot_general(h, c2, _DN,
                                preferred_element_type=jnp.float32)
        o = o + b2[None, :]
        o_ref[:, 0, 0, pl.ds(c * CHUNK, CHUNK)] = jnp.transpose(o, (1, 0))


@functools.partial(jax.jit, static_argnames=())
def kernel(ct_feat, sh_W, sh_b, sh_g, sh_be,
           reg_W1, reg_b1, reg_g1, reg_be1, reg_W2, reg_b2,
           height_W1, height_b1, height_g1, height_be1, height_W2, height_b2,
           dim_W1, dim_b1, dim_g1, dim_be1, dim_W2, dim_b2,
           rot_W1, rot_b1, rot_g1, rot_be1, rot_W2, rot_b2,
           iou_W1, iou_b1, iou_g1, iou_be1, iou_W2, iou_b2,
           hm_W1, hm_b1, hm_g1, hm_be1, hm_W2, hm_b2):
    w1s = (reg_W1, height_W1, dim_W1, rot_W1, iou_W1, hm_W1)
    b1s = (reg_b1, height_b1, dim_b1, rot_b1, iou_b1, hm_b1)
    g1s = (reg_g1, height_g1, dim_g1, rot_g1, iou_g1, hm_g1)
    be1s = (reg_be1, height_be1, dim_be1, rot_be1, iou_be1, hm_be1)
    w2s = (reg_W2, height_W2, dim_W2, rot_W2, iou_W2, hm_W2)
    b2s = (reg_b2, height_b2, dim_b2, rot_b2, iou_b2, hm_b2)

    xt = jnp.swapaxes(ct_feat, 1, 2).reshape(B * N, C_IN)

    full = lambda s: pl.BlockSpec(s, lambda b: tuple(0 for _ in s))
    vec = lambda n: pl.BlockSpec((n,), lambda b: (0,))

    def body(x_ref, shW, shb, shg, shbe,
             w1a, w1b, w1c, w1d, w1e, w1f,
             b1a, b1b, b1c, b1d, b1e, b1f,
             g1a, g1b, g1c, g1d, g1e, g1f,
             be1a, be1b, be1c, be1d, be1e, be1f,
             w2a, w2b, w2c, w2d, w2e, w2f,
             b2a, b2b, b2c, b2d, b2e, b2f,
             o_ref):
        _head_body(x_ref, shW, shb, shg, shbe,
                   (w1a, w1b, w1c, w1d, w1e, w1f),
                   (b1a, b1b, b1c, b1d, b1e, b1f),
                   (g1a, g1b, g1c, g1d, g1e, g1f),
                   (be1a, be1b, be1c, be1d, be1e, be1f),
                   (w2a, w2b, w2c, w2d, w2e, w2f),
                   (b2a, b2b, b2c, b2d, b2e, b2f),
                   o_ref)

    in_specs = ([pl.BlockSpec((N, C_IN), lambda b: (b, 0)),
                 full((C_SH, C_IN)), vec(C_SH), vec(C_SH), vec(C_SH)]
                + [full((C_SH, C_SH))] * 6
                + [vec(C_SH)] * 18
                + [full((cls, C_SH)) for cls in HEAD_CLS]
                + [vec(cls) for cls in HEAD_CLS])

    out = pl.pallas_call(
        body,
        grid=(B,),
        in_specs=in_specs,
        out_specs=pl.BlockSpec((C_OUT, 1, 1, N), lambda b: (0, b, 0, 0)),
        out_shape=jax.ShapeDtypeStruct((C_OUT, B, 1, N), jnp.float32),
        compiler_params=pltpu.CompilerParams(
            dimension_semantics=("parallel",)),
    )(xt, sh_W, sh_b, sh_g, sh_be,
      *w1s, *b1s, *g1s, *be1s, *w2s, *b2s)
    return jnp.swapaxes(out.reshape(C_OUT, B, N), 0, 1)


# speedup vs baseline: 2.4565x; 1.0499x over previous
import functools

import jax
import jax.numpy as jnp
from jax.experimental import pallas as pl
from jax.experimental.pallas import tpu as pltpu

B = 4
C_IN = 256
C_SH = 64
N = 5000
HEAD_CLS = (2, 1, 3, 2, 1, 3)
C_OUT = sum(HEAD_CLS)
C_MID = C_SH * len(HEAD_CLS)
EPS = 1e-5
_DN = (((1,), (1,)), ((), ()))
CHUNK = 625


def _head_body(x_ref, shW_ref, shb_ref, shg_ref, shbe_ref,
               w1_refs, b1_refs, g1_refs, be1_refs, w2_refs, b2_refs,
               o_ref):
    inv_s = 1.0 / jnp.sqrt(1.0 + EPS)

    shW = shW_ref[...].astype(jnp.bfloat16)
    sg = (shg_ref[...] * inv_s)[None, :]
    sb = (shb_ref[...] * shg_ref[...] * inv_s + shbe_ref[...])[None, :]
    w1 = jnp.concatenate([r[...] for r in w1_refs], axis=0).astype(jnp.bfloat16)
    g1 = jnp.concatenate([r[...] for r in g1_refs], axis=0)
    b1 = jnp.concatenate([r[...] for r in b1_refs], axis=0)
    be1 = jnp.concatenate([r[...] for r in be1_refs], axis=0)
    sg1 = (g1 * inv_s)[None, :]
    sb1 = (b1 * g1 * inv_s + be1)[None, :]
    c_rows = []
    for i, r in enumerate(w2_refs):
        c_rows.append(jnp.pad(r[...], ((0, 0), (C_SH * i,
                                                C_MID - C_SH * (i + 1)))))
    c2 = jnp.concatenate(c_rows, axis=0).astype(jnp.bfloat16)
    b2 = jnp.concatenate([r[...] for r in b2_refs], axis=0)

    for c in range(N // CHUNK):
        x = x_ref[pl.ds(c * CHUNK, CHUNK), :].astype(jnp.bfloat16)
        y = jax.lax.dot_general(x, shW, _DN,
                                preferred_element_type=jnp.float32)
        y = jnp.maximum(y * sg + sb, 0.0).astype(jnp.bfloat16)
        h = jax.lax.dot_general(y, w1, _DN,
                                preferred_element_type=jnp.float32)
        h = jnp.maximum(h * sg1 + sb1, 0.0).astype(jnp.bfloat16)
        o = jax.lax.dot_general(h, c2, _DN,
                                preferred_element_type=jnp.float32)
        o = o + b2[None, :]
        o_ref[:, 0, 0, pl.ds(c * CHUNK, CHUNK)] = jnp.transpose(o, (1, 0))


@functools.partial(jax.jit, static_argnames=())
def kernel(ct_feat, sh_W, sh_b, sh_g, sh_be,
           reg_W1, reg_b1, reg_g1, reg_be1, reg_W2, reg_b2,
           height_W1, height_b1, height_g1, height_be1, height_W2, height_b2,
           dim_W1, dim_b1, dim_g1, dim_be1, dim_W2, dim_b2,
           rot_W1, rot_b1, rot_g1, rot_be1, rot_W2, rot_b2,
           iou_W1, iou_b1, iou_g1, iou_be1, iou_W2, iou_b2,
           hm_W1, hm_b1, hm_g1, hm_be1, hm_W2, hm_b2):
    w1s = (reg_W1, height_W1, dim_W1, rot_W1, iou_W1, hm_W1)
    b1s = (reg_b1, height_b1, dim_b1, rot_b1, iou_b1, hm_b1)
    g1s = (reg_g1, height_g1, dim_g1, rot_g1, iou_g1, hm_g1)
    be1s = (reg_be1, height_be1, dim_be1, rot_be1, iou_be1, hm_be1)
    w2s = (reg_W2, height_W2, dim_W2, rot_W2, iou_W2, hm_W2)
    b2s = (reg_b2, height_b2, dim_b2, rot_b2, iou_b2, hm_b2)

    xt = jnp.swapaxes(ct_feat, 1, 2).reshape(B * N, C_IN)

    full = lambda s: pl.BlockSpec(s, lambda b: tuple(0 for _ in s))
    vec = lambda n: pl.BlockSpec((n,), lambda b: (0,))

    def body(x_ref, shW, shb, shg, shbe,
             w1a, w1b, w1c, w1d, w1e, w1f,
             b1a, b1b, b1c, b1d, b1e, b1f,
             g1a, g1b, g1c, g1d, g1e, g1f,
             be1a, be1b, be1c, be1d, be1e, be1f,
             w2a, w2b, w2c, w2d, w2e, w2f,
             b2a, b2b, b2c, b2d, b2e, b2f,
             o_ref):
        _head_body(x_ref, shW, shb, shg, shbe,
                   (w1a, w1b, w1c, w1d, w1e, w1f),
                   (b1a, b1b, b1c, b1d, b1e, b1f),
                   (g1a, g1b, g1c, g1d, g1e, g1f),
                   (be1a, be1b, be1c, be1d, be1e, be1f),
                   (w2a, w2b, w2c, w2d, w2e, w2f),
                   (b2a, b2b, b2c, b2d, b2e, b2f),
                   o_ref)

    in_specs = ([pl.BlockSpec((N, C_IN), lambda b: (b, 0)),
                 full((C_SH, C_IN)), vec(C_SH), vec(C_SH), vec(C_SH)]
                + [full((C_SH, C_SH))] * 6
                + [vec(C_SH)] * 18
                + [full((cls, C_SH)) for cls in HEAD_CLS]
                + [vec(cls) for cls in HEAD_CLS])

    out = pl.pallas_call(
        body,
        grid=(B,),
        in_specs=in_specs,
        out_specs=pl.BlockSpec((C_OUT, 1, 1, N), lambda b: (0, b, 0, 0)),
        out_shape=jax.ShapeDtypeStruct((C_OUT, B, 1, N), jnp.float32),
        compiler_params=pltpu.CompilerParams(
            dimension_semantics=("parallel",)),
    )(xt, sh_W, sh_b, sh_g, sh_be,
      *w1s, *b1s, *g1s, *be1s, *w2s, *b2s)
    return jnp.swapaxes(out.reshape(C_OUT, B, N), 0, 1)
